# baseline (speedup 1.0000x reference)
import math

import jax
import jax.numpy as jnp
from jax.experimental import pallas as pl
from jax.experimental.pallas import tpu as pltpu

L = 10
TILE_C = 2048
BLK = 8 * TILE_C
HALVES = 1
HC = TILE_C // HALVES


def _K(l, m):
    return math.sqrt((2.0 * l + 1.0) / (4.0 * math.pi)
                     * math.factorial(l - m) / math.factorial(l + m))


def _sinpoly(z, w):
    s = 1.5918144e-10
    for c in (-2.5052108e-08, 2.7557319e-06, -1.9841270e-04,
              8.3333333e-03, -1.6666667e-01, 1.0):
        s = s * w + c
    return s * z


def _cospoly(w):
    s = -1.1470746e-11
    for c in (2.0876757e-09, -2.7557319e-07, 2.4801587e-05,
              -1.3888889e-03, 4.1666667e-02, -5.0e-01, 1.0):
        s = s * w + c
    return s


def _store_split(sc_ref, row, lanes, y):
    hi = y.astype(jnp.bfloat16).astype(jnp.float32)
    sc_ref[row, :, lanes] = hi
    sc_ref[row + L * L, :, lanes] = y - hi


def _compute_half(lonh, latn, sc_ref, h):
    lw = latn * latn
    x = _sinpoly(latn, lw)
    sx = _cospoly(lw)
    hw = lonh * lonh
    sh = _sinpoly(lonh, hw)
    ch = _cospoly(hw)
    cp = 2.0 * sh * sh - 1.0
    sp = -2.0 * sh * ch
    two_cp = cp + cp

    cm2, cm1 = None, jnp.ones_like(x)
    sm2, sm1 = None, jnp.zeros_like(x)
    cm = cm1
    sm = sm1
    pmm = jnp.full_like(x, _K(0, 0))

    lanes = pl.ds(h * HC, HC)
    for m in range(L):
        if m == 1:
            cm, sm = cp, sp
        elif m > 1:
            cm, sm = two_cp * cm1 - cm2, two_cp * sm1 - sm2
        if m > 0:
            cm2, cm1 = cm1, cm
            sm2, sm1 = sm1, sm
            f = -(2.0 * m - 1.0) * _K(m, m) / _K(m - 1, m - 1)
            if m == 1:
                f *= math.sqrt(2.0)
            pmm = f * sx * pmm
        p2 = jnp.zeros_like(x)
        p1 = pmm
        for l in range(m, L):
            if l == m:
                p = pmm
            else:
                a = _K(l, m) / _K(l - 1, m) * (2.0 * l - 1.0) / float(l - m)
                b = (-_K(l, m) / _K(l - 2, m) * (l + m - 1.0) / float(l - m)
                     if l >= m + 2 else 0.0)
                p = a * (x * p1) + b * p2
                p2, p1 = p1, p
            if m == 0:
                _store_split(sc_ref, l * l + l, lanes, p)
            else:
                _store_split(sc_ref, l * l + l + m, lanes, cm * p)
                _store_split(sc_ref, l * l + l - m, lanes, sm * p)


def _dot_t(lhs, eye):
    return jax.lax.dot_general(
        lhs, eye,
        dimension_numbers=(((0,), (0,)), ((), ())),
        preferred_element_type=jnp.float32,
        precision=jax.lax.Precision.DEFAULT,
    )


def _emit_dots(sc_ref, out_ref, eye2, h):
    lanes = pl.ds(h * HC, HC)
    for r in range(8):
        out_ref[pl.ds(r * TILE_C + h * HC, HC), :] = _dot_t(
            sc_ref[:, r, lanes], eye2)


def _sh_block(lonh_ref, latn_ref, out_ref, sc_ref):
    eye = jnp.eye(L * L, dtype=jnp.float32)
    eye2 = jnp.concatenate([eye, eye], axis=0)
    for h in range(HALVES):
        lanes = pl.ds(h * HC, HC)
        _compute_half(lonh_ref[:, lanes], latn_ref[:, lanes], sc_ref, h)
        _emit_dots(sc_ref, out_ref, eye2, h)


def kernel(lonlat):
    n = lonlat.shape[0]
    nblk = pl.cdiv(n, BLK)
    npad = nblk * BLK
    lonh = jnp.pad(lonlat[:, 0] * (math.pi / 360.0), (0, npad - n)).reshape(-1, TILE_C)
    latn = jnp.pad(lonlat[:, 1] * (-math.pi / 180.0), (0, npad - n)).reshape(-1, TILE_C)
    return pl.pallas_call(
        _sh_block,
        grid=(nblk,),
        in_specs=[
            pl.BlockSpec((8, TILE_C), lambda i: (i, 0)),
            pl.BlockSpec((8, TILE_C), lambda i: (i, 0)),
        ],
        out_specs=pl.BlockSpec((BLK, L * L), lambda i: (i, 0)),
        out_shape=jax.ShapeDtypeStruct((n, L * L), jnp.float32),
        scratch_shapes=[pltpu.VMEM((2 * L * L, 8, TILE_C), jnp.float32)],
    )(lonh, latn)


# speedup vs baseline: 1.2977x; 1.2977x over previous
import math

import jax
import jax.numpy as jnp
from jax.experimental import pallas as pl
from jax.experimental.pallas import tpu as pltpu

L = 10
TILE_C = 2048
BLK = 8 * TILE_C
HALVES = 2
HC = TILE_C // HALVES


def _K(l, m):
    return math.sqrt((2.0 * l + 1.0) / (4.0 * math.pi)
                     * math.factorial(l - m) / math.factorial(l + m))


def _sinpoly(z, w):
    s = 1.5918144e-10
    for c in (-2.5052108e-08, 2.7557319e-06, -1.9841270e-04,
              8.3333333e-03, -1.6666667e-01, 1.0):
        s = s * w + c
    return s * z


def _cospoly(w):
    s = -1.1470746e-11
    for c in (2.0876757e-09, -2.7557319e-07, 2.4801587e-05,
              -1.3888889e-03, 4.1666667e-02, -5.0e-01, 1.0):
        s = s * w + c
    return s


def _compute_half(lonh, latn, sc_ref, h):
    lw = latn * latn
    x = _sinpoly(latn, lw)
    sx = _cospoly(lw)
    hw = lonh * lonh
    sh = _sinpoly(lonh, hw)
    ch = _cospoly(hw)
    cp = 2.0 * sh * sh - 1.0
    sp = -2.0 * sh * ch
    two_cp = cp + cp

    cm2, cm1 = None, jnp.ones_like(x)
    sm2, sm1 = None, jnp.zeros_like(x)
    cm = cm1
    sm = sm1
    pmm = jnp.full_like(x, _K(0, 0))

    lanes = pl.ds(h * HC, HC)
    for m in range(L):
        if m == 1:
            cm, sm = cp, sp
        elif m > 1:
            cm, sm = two_cp * cm1 - cm2, two_cp * sm1 - sm2
        if m > 0:
            cm2, cm1 = cm1, cm
            sm2, sm1 = sm1, sm
            f = -(2.0 * m - 1.0) * _K(m, m) / _K(m - 1, m - 1)
            if m == 1:
                f *= math.sqrt(2.0)
            pmm = f * sx * pmm
        p2 = jnp.zeros_like(x)
        p1 = pmm
        for l in range(m, L):
            if l == m:
                p = pmm
            else:
                a = _K(l, m) / _K(l - 1, m) * (2.0 * l - 1.0) / float(l - m)
                b = (-_K(l, m) / _K(l - 2, m) * (l + m - 1.0) / float(l - m)
                     if l >= m + 2 else 0.0)
                p = a * (x * p1) + b * p2
                p2, p1 = p1, p
            if m == 0:
                sc_ref[l * l + l, :, lanes] = p
            else:
                sc_ref[l * l + l + m, :, lanes] = cm * p
                sc_ref[l * l + l - m, :, lanes] = sm * p


def _emit_dots(sc_ref, out_ref, eye, h):
    for r in range(8):
        out_ref[pl.ds(r * TILE_C + h * HC, HC), :] = jax.lax.dot_general(
            sc_ref[:, r, pl.ds(h * HC, HC)], eye,
            dimension_numbers=(((0,), (0,)), ((), ())),
            preferred_element_type=jnp.float32,
            precision=jax.lax.Precision.DEFAULT,
        )


def _sh_block(lonh_ref, latn_ref, out_ref, sc_ref):
    eye = jnp.eye(L * L, dtype=jnp.float32)
    for h in range(HALVES):
        lanes = pl.ds(h * HC, HC)
        _compute_half(lonh_ref[:, lanes], latn_ref[:, lanes], sc_ref, h)
        _emit_dots(sc_ref, out_ref, eye, h)


def kernel(lonlat):
    n = lonlat.shape[0]
    nblk = pl.cdiv(n, BLK)
    npad = nblk * BLK
    lonh = jnp.pad(lonlat[:, 0] * (math.pi / 360.0), (0, npad - n)).reshape(-1, TILE_C)
    latn = jnp.pad(lonlat[:, 1] * (-math.pi / 180.0), (0, npad - n)).reshape(-1, TILE_C)
    return pl.pallas_call(
        _sh_block,
        grid=(nblk,),
        in_specs=[
            pl.BlockSpec((8, TILE_C), lambda i: (i, 0)),
            pl.BlockSpec((8, TILE_C), lambda i: (i, 0)),
        ],
        out_specs=pl.BlockSpec((BLK, L * L), lambda i: (i, 0)),
        out_shape=jax.ShapeDtypeStruct((n, L * L), jnp.float32),
        scratch_shapes=[pltpu.VMEM((L * L, 8, TILE_C), jnp.float32)],
    )(lonh, latn)


# speedup vs baseline: 1.2993x; 1.0013x over previous
import math

import jax
import jax.numpy as jnp
from jax.experimental import pallas as pl
from jax.experimental.pallas import tpu as pltpu

L = 10
TILE_C = 4096
BLK = 8 * TILE_C
HALVES = 2
HC = TILE_C // HALVES


def _K(l, m):
    return math.sqrt((2.0 * l + 1.0) / (4.0 * math.pi)
                     * math.factorial(l - m) / math.factorial(l + m))


def _sinpoly(z, w):
    s = 1.5918144e-10
    for c in (-2.5052108e-08, 2.7557319e-06, -1.9841270e-04,
              8.3333333e-03, -1.6666667e-01, 1.0):
        s = s * w + c
    return s * z


def _cospoly(w):
    s = -1.1470746e-11
    for c in (2.0876757e-09, -2.7557319e-07, 2.4801587e-05,
              -1.3888889e-03, 4.1666667e-02, -5.0e-01, 1.0):
        s = s * w + c
    return s


def _compute_half(lonh, latn, sc_ref, h):
    lw = latn * latn
    x = _sinpoly(latn, lw)
    sx = _cospoly(lw)
    hw = lonh * lonh
    sh = _sinpoly(lonh, hw)
    ch = _cospoly(hw)
    cp = 2.0 * sh * sh - 1.0
    sp = -2.0 * sh * ch
    two_cp = cp + cp

    cm2, cm1 = None, jnp.ones_like(x)
    sm2, sm1 = None, jnp.zeros_like(x)
    cm = cm1
    sm = sm1
    pmm = jnp.full_like(x, _K(0, 0))

    lanes = pl.ds(h * HC, HC)
    for m in range(L):
        if m == 1:
            cm, sm = cp, sp
        elif m > 1:
            cm, sm = two_cp * cm1 - cm2, two_cp * sm1 - sm2
        if m > 0:
            cm2, cm1 = cm1, cm
            sm2, sm1 = sm1, sm
            f = -(2.0 * m - 1.0) * _K(m, m) / _K(m - 1, m - 1)
            if m == 1:
                f *= math.sqrt(2.0)
            pmm = f * sx * pmm
        p2 = jnp.zeros_like(x)
        p1 = pmm
        for l in range(m, L):
            if l == m:
                p = pmm
            else:
                a = _K(l, m) / _K(l - 1, m) * (2.0 * l - 1.0) / float(l - m)
                b = (-_K(l, m) / _K(l - 2, m) * (l + m - 1.0) / float(l - m)
                     if l >= m + 2 else 0.0)
                p = a * (x * p1) + b * p2
                p2, p1 = p1, p
            if m == 0:
                sc_ref[l * l + l, :, lanes] = p
            else:
                sc_ref[l * l + l + m, :, lanes] = cm * p
                sc_ref[l * l + l - m, :, lanes] = sm * p


def _emit_dots(sc_ref, out_ref, eye, h):
    for r in range(8):
        out_ref[pl.ds(r * TILE_C + h * HC, HC), :] = jax.lax.dot_general(
            sc_ref[:, r, pl.ds(h * HC, HC)], eye,
            dimension_numbers=(((0,), (0,)), ((), ())),
            preferred_element_type=jnp.float32,
            precision=jax.lax.Precision.DEFAULT,
        )


def _sh_block(lonh_ref, latn_ref, out_ref, sc_ref):
    eye = jnp.eye(L * L, dtype=jnp.float32)
    for h in range(HALVES):
        lanes = pl.ds(h * HC, HC)
        _compute_half(lonh_ref[:, lanes], latn_ref[:, lanes], sc_ref, h)
        _emit_dots(sc_ref, out_ref, eye, h)


def kernel(lonlat):
    n = lonlat.shape[0]
    nblk = pl.cdiv(n, BLK)
    npad = nblk * BLK
    lonh = jnp.pad(lonlat[:, 0] * (math.pi / 360.0), (0, npad - n)).reshape(-1, TILE_C)
    latn = jnp.pad(lonlat[:, 1] * (-math.pi / 180.0), (0, npad - n)).reshape(-1, TILE_C)
    return pl.pallas_call(
        _sh_block,
        grid=(nblk,),
        in_specs=[
            pl.BlockSpec((8, TILE_C), lambda i: (i, 0)),
            pl.BlockSpec((8, TILE_C), lambda i: (i, 0)),
        ],
        out_specs=pl.BlockSpec((BLK, L * L), lambda i: (i, 0)),
        out_shape=jax.ShapeDtypeStruct((n, L * L), jnp.float32),
        scratch_shapes=[pltpu.VMEM((L * L, 8, TILE_C), jnp.float32)],
    )(lonh, latn)
